# baseline (speedup 1.0000x reference)
import functools

import jax
import jax.numpy as jnp
from jax import lax
from jax.experimental import pallas as pl
from jax.experimental.pallas import tpu as pltpu
from jax.experimental.pallas import tpu_sc as plsc

NC = 2
NS = 16
NW = NC * NS
LANES = 16


def _make_emb_kernel(B, S, D, N, SW, C, SCH):
    mesh = plsc.VectorSubcoreMesh(
        core_axis_name="c", subcore_axis_name="s",
        num_cores=NC, num_subcores=NS,
    )

    @functools.partial(
        pl.kernel,
        mesh=mesh,
        out_type=jax.ShapeDtypeStruct((N, D), jnp.float32),
        scratch_types=[
            pltpu.VMEM((B, SW), jnp.int32),
            pltpu.VMEM((C, D), jnp.float32),
            pltpu.VMEM((C, D), jnp.float32),
            pltpu.SemaphoreType.DMA,
        ],
    )
    def emb_kernel(idx_hbm, pe_hbm, table_hbm, out_hbm, idx_v, pe_v, rows_v,
                   sem):
        wid = lax.axis_index("s") * NC + lax.axis_index("c")
        s_base = wid * SW
        pltpu.sync_copy(idx_hbm.at[wid], idx_v)
        for sc in range(SCH):
            p0 = pl.multiple_of(s_base + sc * C, 8)
            pltpu.sync_copy(pe_hbm.at[pl.ds(p0, C)], pe_v)
            for b in range(B):
                pltpu.async_copy(
                    table_hbm.at[idx_v.at[b, pl.ds(sc * C, C)]], rows_v, sem
                ).wait()

                def row_body(r, carry):
                    for k in range(D // LANES):
                        sl = pl.ds(k * LANES, LANES)
                        plsc.addupdate(rows_v.at[r, sl], pe_v[r, sl])
                    return carry

                lax.fori_loop(0, C, row_body, 0)
                r0 = pl.multiple_of(b * S + s_base + sc * C, 8)
                pltpu.sync_copy(rows_v, out_hbm.at[pl.ds(r0, C)])

    return emb_kernel


def kernel(x, table, pos_encoding):
    B, S = x.shape
    D = table.shape[1]
    N = B * S
    SW = S // NW
    C = 64
    SCH = SW // C

    idx = x.astype(jnp.int32).reshape(B, NW, SW).transpose(1, 0, 2)
    pe = pos_encoding.reshape(S, D).astype(jnp.float32)

    emb = _make_emb_kernel(B, S, D, N, SW, C, SCH)
    out = emb(idx, pe, table)
    return out.reshape(B, S, D)
